# baseline (speedup 1.0000x reference)
import functools

import jax
import jax.numpy as jnp
from jax import lax
from jax.experimental import pallas as pl
from jax.experimental.pallas import tpu as pltpu
from jax.experimental.pallas import tpu_sc as plsc

EMB = 32
L = 16
NC = 2
NS = 16
NW = NC * NS


def _sc_body(batch, row_id, col_id, row_emb, col_emb, row_bias, col_bias,
             gb16, out_hbm, ridx_v, cidx_v, rbuf, cbuf, rbb, cbb, gb_v,
             out_v, sem):
    bpw = batch // NW
    ngroups = bpw // L
    wid = lax.axis_index("s") * NC + lax.axis_index("c")
    base = wid * bpw

    pltpu.sync_copy(row_id.at[pl.ds(base, bpw)], ridx_v)
    pltpu.sync_copy(col_id.at[pl.ds(base, bpw)], cidx_v)
    pltpu.sync_copy(gb16, gb_v)

    gvec = gb_v[...]
    lane = lax.iota(jnp.int32, L)
    zero16 = jnp.zeros((L,), jnp.int32)

    def fire(g, slot):
        iv = ridx_v[pl.ds(g * L, L)]
        cv = cidx_v[pl.ds(g * L, L)]
        for k in range(L):
            rk = iv[k]
            ck = cv[k]
            d = pl.ds(slot + k, 1)
            pltpu.async_copy(row_emb.at[pl.ds(rk, 1)], rbuf.at[d], sem)
            pltpu.async_copy(col_emb.at[pl.ds(ck, 1)], cbuf.at[d], sem)
            pltpu.async_copy(row_bias.at[pl.ds(rk, 1)], rbb.at[d], sem)
            pltpu.async_copy(col_bias.at[pl.ds(ck, 1)], cbb.at[d], sem)

    fire(0, 0)

    def step(g, _):
        slot = lax.rem(g, 2) * L
        sl = pl.ds(slot, L)
        pltpu.make_async_copy(row_emb.at[pl.ds(0, L)], rbuf.at[sl], sem).wait()
        pltpu.make_async_copy(col_emb.at[pl.ds(0, L)], cbuf.at[sl], sem).wait()
        pltpu.make_async_copy(row_bias.at[pl.ds(0, L)], rbb.at[sl], sem).wait()
        pltpu.make_async_copy(col_bias.at[pl.ds(0, L)], cbb.at[sl], sem).wait()

        @pl.when(g + 1 < ngroups)
        def _():
            fire(g + 1, (L - slot))

        idx_b = lane + slot
        acc = (plsc.load_gather(rbb, [idx_b, zero16])
               + plsc.load_gather(cbb, [idx_b, zero16]) + gvec)
        for e in range(EMB):
            e_idx = jnp.full((L,), e, jnp.int32)
            rv = plsc.load_gather(rbuf, [idx_b, e_idx])
            cv = plsc.load_gather(cbuf, [idx_b, e_idx])
            acc += rv * cv
        out_v[pl.ds(g * L, L)] = acc
        return ()

    lax.fori_loop(0, ngroups, step, ())

    pltpu.sync_copy(out_v, out_hbm.at[pl.ds(base, bpw)])


@functools.partial(jax.jit, static_argnames=("batch",))
def _mf_sc(row_id, col_id, row_emb, col_emb, row_bias, col_bias, gb16,
           *, batch):
    bpw = batch // NW
    mesh = plsc.VectorSubcoreMesh(core_axis_name="c", subcore_axis_name="s")
    return pl.kernel(
        functools.partial(_sc_body, batch),
        out_type=jax.ShapeDtypeStruct((batch,), jnp.float32),
        mesh=mesh,
        scratch_types=[
            pltpu.VMEM((bpw,), jnp.int32),
            pltpu.VMEM((bpw,), jnp.int32),
            pltpu.VMEM((2 * L, EMB), jnp.float32),
            pltpu.VMEM((2 * L, EMB), jnp.float32),
            pltpu.VMEM((2 * L, 1), jnp.float32),
            pltpu.VMEM((2 * L, 1), jnp.float32),
            pltpu.VMEM((L,), jnp.float32),
            pltpu.VMEM((bpw,), jnp.float32),
            pltpu.SemaphoreType.DMA,
        ],
        compiler_params=pltpu.CompilerParams(needs_layout_passes=False),
    )(row_id, col_id, row_emb, col_emb, row_bias, col_bias, gb16)


def kernel(row_id, col_id, row_emb_table, col_emb_table, row_bias_table,
           col_bias_table, global_bias):
    batch = row_id.shape[0]
    gb16 = jnp.broadcast_to(jnp.reshape(global_bias, (1,)), (L,))
    out = _mf_sc(row_id.astype(jnp.int32), col_id.astype(jnp.int32),
                 row_emb_table, col_emb_table, row_bias_table,
                 col_bias_table, gb16, batch=batch)
    return out.reshape(batch, 1)


# speedup vs baseline: 1.0110x; 1.0110x over previous
import functools

import jax
import jax.numpy as jnp
from jax import lax
from jax.experimental import pallas as pl
from jax.experimental.pallas import tpu as pltpu
from jax.experimental.pallas import tpu_sc as plsc

EMB = 32
L = 16
NC = 2
NS = 16
NW = NC * NS
DEPTH = 4


def _sc_body(batch, row_id, col_id, row_emb, col_emb, row_bias, col_bias,
             gb16, out_hbm, ridx_v, cidx_v, rbuf, cbuf, rbb, cbb, gb_v,
             out_v, sem):
    bpw = batch // NW
    ngroups = bpw // L
    wid = lax.axis_index("s") * NC + lax.axis_index("c")
    base = wid * bpw

    pltpu.sync_copy(row_id.at[pl.ds(base, bpw)], ridx_v)
    pltpu.sync_copy(col_id.at[pl.ds(base, bpw)], cidx_v)
    pltpu.sync_copy(gb16, gb_v)

    gvec = gb_v[...]
    lane = lax.iota(jnp.int32, L)
    zero16 = jnp.zeros((L,), jnp.int32)

    def fire(g, slot):
        iv = ridx_v[pl.ds(g * L, L)]
        cv = cidx_v[pl.ds(g * L, L)]
        for k in range(L):
            rk = iv[k]
            ck = cv[k]
            d = pl.ds(slot + k, 1)
            pltpu.async_copy(row_emb.at[pl.ds(rk, 1)], rbuf.at[d], sem)
            pltpu.async_copy(col_emb.at[pl.ds(ck, 1)], cbuf.at[d], sem)
            pltpu.async_copy(row_bias.at[pl.ds(rk, 1)], rbb.at[d], sem)
            pltpu.async_copy(col_bias.at[pl.ds(ck, 1)], cbb.at[d], sem)

    for p in range(DEPTH - 1):
        fire(p, p * L)

    def step(g, _):
        slot = lax.rem(g, DEPTH) * L
        sl = pl.ds(slot, L)
        pltpu.make_async_copy(row_emb.at[pl.ds(0, L)], rbuf.at[sl], sem).wait()
        pltpu.make_async_copy(col_emb.at[pl.ds(0, L)], cbuf.at[sl], sem).wait()
        pltpu.make_async_copy(row_bias.at[pl.ds(0, L)], rbb.at[sl], sem).wait()
        pltpu.make_async_copy(col_bias.at[pl.ds(0, L)], cbb.at[sl], sem).wait()

        @pl.when(g + DEPTH - 1 < ngroups)
        def _():
            fire(g + DEPTH - 1, lax.rem(g + DEPTH - 1, DEPTH) * L)

        idx_b = lane + slot
        acc = (plsc.load_gather(rbb, [idx_b, zero16])
               + plsc.load_gather(cbb, [idx_b, zero16]) + gvec)
        for e in range(EMB):
            e_idx = jnp.full((L,), e, jnp.int32)
            rv = plsc.load_gather(rbuf, [idx_b, e_idx])
            cv = plsc.load_gather(cbuf, [idx_b, e_idx])
            acc += rv * cv
        out_v[pl.ds(g * L, L)] = acc
        return ()

    lax.fori_loop(0, ngroups, step, ())

    pltpu.sync_copy(out_v, out_hbm.at[pl.ds(base, bpw)])


@functools.partial(jax.jit, static_argnames=("batch",))
def _mf_sc(row_id, col_id, row_emb, col_emb, row_bias, col_bias, gb16,
           *, batch):
    bpw = batch // NW
    mesh = plsc.VectorSubcoreMesh(core_axis_name="c", subcore_axis_name="s")
    return pl.kernel(
        functools.partial(_sc_body, batch),
        out_type=jax.ShapeDtypeStruct((batch,), jnp.float32),
        mesh=mesh,
        scratch_types=[
            pltpu.VMEM((bpw,), jnp.int32),
            pltpu.VMEM((bpw,), jnp.int32),
            pltpu.VMEM((DEPTH * L, EMB), jnp.float32),
            pltpu.VMEM((DEPTH * L, EMB), jnp.float32),
            pltpu.VMEM((DEPTH * L, 1), jnp.float32),
            pltpu.VMEM((DEPTH * L, 1), jnp.float32),
            pltpu.VMEM((L,), jnp.float32),
            pltpu.VMEM((bpw,), jnp.float32),
            pltpu.SemaphoreType.DMA,
        ],
        compiler_params=pltpu.CompilerParams(needs_layout_passes=False),
    )(row_id, col_id, row_emb, col_emb, row_bias, col_bias, gb16)


def kernel(row_id, col_id, row_emb_table, col_emb_table, row_bias_table,
           col_bias_table, global_bias):
    batch = row_id.shape[0]
    gb16 = jnp.broadcast_to(jnp.reshape(global_bias, (1,)), (L,))
    out = _mf_sc(row_id.astype(jnp.int32), col_id.astype(jnp.int32),
                 row_emb_table, col_emb_table, row_bias_table,
                 col_bias_table, gb16, batch=batch)
    return out.reshape(batch, 1)
